# baseline (speedup 1.0000x reference)
import functools

import jax
import jax.numpy as jnp
import numpy as np
from jax import lax
from jax.experimental import pallas as pl
from jax.experimental.pallas import tpu as pltpu
from jax.experimental.pallas import tpu_sc as plsc

_R_CUT = 5.0
_N_POINTS = 600
_N_MAX_L = (20, 18, 16, 14)
_OFFS = (0, 20, 38, 54, 68)
_TOTAL_N = 68
_HID = 32

_NC, _NSUB = 2, 16
_NW = _NC * _NSUB
_CHUNK = 128
_ROW = 256
_BLK = 640

_SCALE = np.float32((_N_POINTS - 1) / _R_CUT)
_H = np.float32(_R_CUT / (_N_POINTS - 1))


def _sc_gather(r, ftab):
    n = r.shape[0]
    per_w = n // _NW
    n_chunks = (per_w + _CHUNK - 1) // _CHUNK
    last_off = per_w - _CHUNK
    mesh = plsc.VectorSubcoreMesh(core_axis_name="c", subcore_axis_name="s")

    @functools.partial(
        pl.kernel,
        out_type=jax.ShapeDtypeStruct((2, n, 128), jnp.int32),
        mesh=mesh,
        scratch_types=[
            pltpu.VMEM((_CHUNK,), jnp.float32),
            pltpu.VMEM((_CHUNK,), jnp.int32),
            pltpu.VMEM((_CHUNK,), jnp.int32),
            pltpu.VMEM((_CHUNK, _ROW), jnp.int32),
            pltpu.VMEM((_CHUNK, _ROW), jnp.int32),
            pltpu.SemaphoreType.DMA,
            pltpu.SemaphoreType.DMA,
        ],
    )
    def k(r_hbm, tab_hbm, out_hbm, r_v, idx0, idx1, rows0, rows1, sem0, sem1):
        wid = lax.axis_index("s") * _NC + lax.axis_index("c")
        base = wid * per_w

        def chunk_off(c):
            return base + jnp.minimum(c * _CHUNK, last_off)

        def load_idx(c, idx_v):
            off = chunk_off(c)
            pltpu.sync_copy(r_hbm.at[pl.ds(off, _CHUNK)], r_v)
            for kk in range(_CHUNK // 16):
                rv = r_v[pl.ds(kk * 16, 16)]
                i0 = (rv * _SCALE).astype(jnp.int32)
                i0 = jnp.minimum(jnp.maximum(i0, 0), _N_POINTS - 2)
                idx_v[pl.ds(kk * 16, 16)] = i0

        bufs = ((idx0, rows0, sem0), (idx1, rows1, sem1))

        for b in range(2):
            idx_v, rows_v, sem = bufs[b]
            load_idx(b, idx_v)
            pltpu.async_copy(tab_hbm.at[idx_v], rows_v, sem)

        def body(i, carry):
            for b in range(2):
                c = 2 * i + b
                idx_v, rows_v, sem = bufs[b]
                pltpu.make_async_copy(tab_hbm.at[idx_v], rows_v, sem).wait()
                off = chunk_off(c)
                pltpu.sync_copy(rows_v.at[:, pl.ds(0, 128)],
                                out_hbm.at[0, pl.ds(off, _CHUNK)])
                pltpu.sync_copy(rows_v.at[:, pl.ds(128, 128)],
                                out_hbm.at[1, pl.ds(off, _CHUNK)])

                @pl.when(c + 2 < n_chunks)
                def _():
                    load_idx(c + 2, idx_v)
                    pltpu.async_copy(tab_hbm.at[idx_v], rows_v, sem)

            return carry

        lax.fori_loop(0, (n_chunks + 1) // 2, body, 0)

    return k(r, ftab)


def _tc_body(g_ref, r_ref, s_ref, w1_ref, w23_ref, w4_ref, comb_ref,
             o0, o1, o2, o3):
    gw0 = g_ref[0]
    gw1 = g_ref[1]
    mask = jnp.int32(-65536)

    def unpack(gw):
        lo = lax.bitcast_convert_type(jnp.left_shift(gw, 16), jnp.float32)
        hi = lax.bitcast_convert_type(jnp.bitwise_and(gw, mask), jnp.float32)
        return lo, hi

    v0, v1 = unpack(gw0)
    d0, d1 = unpack(gw1)
    t = r_ref[...] * _SCALE
    i0 = t.astype(jnp.int32)
    i0 = jnp.minimum(jnp.maximum(i0, 0), _N_POINTS - 2)
    f = t - i0.astype(jnp.float32)
    f2 = f * f
    f3 = f2 * f
    h00 = 2.0 * f3 - 3.0 * f2 + 1.0
    h10 = _H * (f3 - 2.0 * f2 + f)
    h01 = -2.0 * f3 + 3.0 * f2
    h11 = _H * (f3 - f2)
    radial = h00 * v0 + h10 * d0 + h01 * v1 + h11 * d1

    s = s_ref[...]
    pa = []
    for aj in range(4):
        v = jnp.where(s == 0, comb_ref[aj, 0],
            jnp.where(s == 1, comb_ref[aj, 1],
            jnp.where(s == 2, comb_ref[aj, 2], comb_ref[aj, 3])))
        pa.append(v)

    z = jnp.dot(radial, w1_ref[...], preferred_element_type=jnp.float32)
    lane = lax.broadcasted_iota(jnp.int32, (z.shape[0], 128), 1)
    ajidx = lane // _HID
    pat = jnp.where(ajidx == 0, pa[0],
          jnp.where(ajidx == 1, pa[1],
          jnp.where(ajidx == 2, pa[2], pa[3])))

    def silu(x):
        xh = 0.5 * x
        return xh * jnp.tanh(xh) + xh

    outs = (o0, o1, o2, o3)
    for l in range(4):
        n = _N_MAX_L[l]
        hh = silu(z[:, l * 128:(l + 1) * 128] * pat)
        for layer in range(2):
            y = jnp.dot(hh, w23_ref[layer, l], preferred_element_type=jnp.float32)
            hh = silu(y)
        y = jnp.dot(hh, w4_ref[l][:, :4 * n], preferred_element_type=jnp.float32)
        outs[l][...] = y.reshape(y.shape[0], 4, n)


def _tc_mlp(G, r2, sp2, W1all, W23, W4all, comb_W):
    npad = G.shape[1]
    nb = npad // _BLK
    out_shapes = tuple(jax.ShapeDtypeStruct((npad, 4, n), jnp.float32)
                       for n in _N_MAX_L)
    return pl.pallas_call(
        _tc_body,
        grid=(nb,),
        in_specs=[
            pl.BlockSpec((2, _BLK, 128), lambda i: (0, i, 0)),
            pl.BlockSpec((_BLK, 1), lambda i: (i, 0)),
            pl.BlockSpec((_BLK, 1), lambda i: (i, 0)),
            pl.BlockSpec((128, 512), lambda i: (0, 0)),
            pl.BlockSpec((2, 4, 128, 128), lambda i: (0, 0, 0, 0)),
            pl.BlockSpec((4, 128, 80), lambda i: (0, 0, 0)),
            pl.BlockSpec(memory_space=pltpu.SMEM),
        ],
        out_specs=tuple(pl.BlockSpec((_BLK, 4, n), lambda i: (i, 0, 0))
                        for n in _N_MAX_L),
        out_shape=out_shapes,
    )(G, r2, sp2, W1all, W23, W4all, comb_W)


def _pack_weights(mlp_params):
    eye4 = jnp.eye(4, dtype=jnp.float32)
    w1_cols, w23_l, w4_l = [], [[], []], []
    for l in range(4):
        n = _N_MAX_L[l]
        o = _OFFS[l]
        p = [mlp_params[str(l) + '_' + str(aj)] for aj in range(4)]
        w1 = jnp.stack([pi[0] for pi in p])
        w1 = w1.transpose(2, 0, 1).reshape(n, 128)
        w1_cols.append(jnp.pad(w1, ((o, 128 - o - n), (0, 0))))
        for layer in range(2):
            w = jnp.stack([pi[1 + layer].T for pi in p])
            bd = jnp.einsum('aij,ab->aibj', w, eye4).reshape(128, 128)
            w23_l[layer].append(bd)
        w4 = jnp.stack([pi[3].T for pi in p])
        bd = jnp.einsum('aij,ab->aibj', w4, eye4).reshape(128, 4 * n)
        w4_l.append(jnp.pad(bd, ((0, 0), (0, 80 - 4 * n))))
    W1all = jnp.concatenate(w1_cols, axis=1)
    W23 = jnp.stack([jnp.stack(w23_l[0]), jnp.stack(w23_l[1])])
    W4all = jnp.stack(w4_l)
    return W1all, W23, W4all


def _fused_table(spline_values, spline_derivs):
    def slab(x):
        xb = x.astype(jnp.bfloat16)
        u = lax.bitcast_convert_type(xb, jnp.uint16).astype(jnp.uint32)
        return jnp.pad(u, ((0, 0), (0, 128 - _TOTAL_N)))

    sv1 = jnp.concatenate([spline_values[1:], spline_values[:1]], axis=0)
    sd1 = jnp.concatenate([spline_derivs[1:], spline_derivs[:1]], axis=0)
    lo = jnp.concatenate([slab(spline_values), slab(spline_derivs)], axis=1)
    hi = jnp.concatenate([slab(sv1), slab(sd1)], axis=1)
    return lax.bitcast_convert_type((hi << 16) | lo, jnp.int32)


def kernel(r, species_neighbor_idx, spline_values, spline_derivs, comb_W,
           mlp_params):
    n = r.shape[0]
    ftab = _fused_table(spline_values, spline_derivs)
    W1all, W23, W4all = _pack_weights(mlp_params)
    G = _sc_gather(r, ftab)
    return _tc_mlp(G, r.reshape(n, 1), species_neighbor_idx.reshape(n, 1),
                   W1all, W23, W4all, comb_W)


# speedup vs baseline: 1.3965x; 1.3965x over previous
import functools

import jax
import jax.numpy as jnp
import numpy as np
from jax import lax
from jax.experimental import pallas as pl
from jax.experimental.pallas import tpu as pltpu
from jax.experimental.pallas import tpu_sc as plsc

_R_CUT = 5.0
_N_POINTS = 600
_N_MAX_L = (20, 18, 16, 14)
_OFFS = (0, 20, 38, 54, 68)
_TOTAL_N = 68
_HID = 32

_NC, _NSUB = 2, 16
_NW = _NC * _NSUB
_CHUNK = 128
_ROW = 256
_BLK = 640

_SCALE = np.float32((_N_POINTS - 1) / _R_CUT)
_H = np.float32(_R_CUT / (_N_POINTS - 1))


def _sc_gather(r, ftab):
    n = r.shape[0]
    per_w = n // _NW
    n_chunks = (per_w + _CHUNK - 1) // _CHUNK
    last_off = per_w - _CHUNK
    mesh = plsc.VectorSubcoreMesh(core_axis_name="c", subcore_axis_name="s")

    @functools.partial(
        pl.kernel,
        out_type=jax.ShapeDtypeStruct((2, n, 128), jnp.int32),
        mesh=mesh,
        scratch_types=[
            pltpu.VMEM((_CHUNK,), jnp.float32),
            pltpu.VMEM((_CHUNK,), jnp.int32),
            pltpu.VMEM((_CHUNK,), jnp.int32),
            pltpu.VMEM((_CHUNK, _ROW), jnp.int32),
            pltpu.VMEM((_CHUNK, _ROW), jnp.int32),
            pltpu.SemaphoreType.DMA,
            pltpu.SemaphoreType.DMA,
        ],
    )
    def k(r_hbm, tab_hbm, out_hbm, r_v, idx0, idx1, rows0, rows1, sem0, sem1):
        wid = lax.axis_index("s") * _NC + lax.axis_index("c")
        base = wid * per_w

        def chunk_off(c):
            return base + jnp.minimum(c * _CHUNK, last_off)

        def load_idx(c, idx_v):
            off = chunk_off(c)
            pltpu.sync_copy(r_hbm.at[pl.ds(off, _CHUNK)], r_v)
            for kk in range(_CHUNK // 16):
                rv = r_v[pl.ds(kk * 16, 16)]
                i0 = (rv * _SCALE).astype(jnp.int32)
                i0 = jnp.minimum(jnp.maximum(i0, 0), _N_POINTS - 2)
                idx_v[pl.ds(kk * 16, 16)] = i0

        bufs = ((idx0, rows0, sem0), (idx1, rows1, sem1))

        for b in range(2):
            idx_v, rows_v, sem = bufs[b]
            load_idx(b, idx_v)
            pltpu.async_copy(tab_hbm.at[idx_v], rows_v, sem)

        def body(i, carry):
            for b in range(2):
                c = 2 * i + b
                idx_v, rows_v, sem = bufs[b]
                pltpu.make_async_copy(tab_hbm.at[idx_v], rows_v, sem).wait()
                off = chunk_off(c)
                pltpu.sync_copy(rows_v.at[:, pl.ds(0, 128)],
                                out_hbm.at[0, pl.ds(off, _CHUNK)])
                pltpu.sync_copy(rows_v.at[:, pl.ds(128, 128)],
                                out_hbm.at[1, pl.ds(off, _CHUNK)])

                @pl.when(c + 2 < n_chunks)
                def _():
                    load_idx(c + 2, idx_v)
                    pltpu.async_copy(tab_hbm.at[idx_v], rows_v, sem)

            return carry

        lax.fori_loop(0, (n_chunks + 1) // 2, body, 0)

    return k(r, ftab)


def _tc_body(g_ref, r_ref, s_ref, w1_ref, w23_ref, w4_ref, comb_ref,
             o0, o1, o2, o3):
    gw0 = g_ref[0]
    gw1 = g_ref[1]
    mask = jnp.int32(-65536)

    def unpack(gw):
        lo = lax.bitcast_convert_type(jnp.left_shift(gw, 16), jnp.float32)
        hi = lax.bitcast_convert_type(jnp.bitwise_and(gw, mask), jnp.float32)
        return lo, hi

    v0, v1 = unpack(gw0)
    d0, d1 = unpack(gw1)
    t = r_ref[...] * _SCALE
    i0 = t.astype(jnp.int32)
    i0 = jnp.minimum(jnp.maximum(i0, 0), _N_POINTS - 2)
    f = t - i0.astype(jnp.float32)
    f2 = f * f
    f3 = f2 * f
    h00 = 2.0 * f3 - 3.0 * f2 + 1.0
    h10 = _H * (f3 - 2.0 * f2 + f)
    h01 = -2.0 * f3 + 3.0 * f2
    h11 = _H * (f3 - f2)
    radial = h00 * v0 + h10 * d0 + h01 * v1 + h11 * d1

    s = s_ref[...]
    pa = []
    for aj in range(4):
        v = jnp.where(s == 0, comb_ref[aj, 0],
            jnp.where(s == 1, comb_ref[aj, 1],
            jnp.where(s == 2, comb_ref[aj, 2], comb_ref[aj, 3])))
        pa.append(v)

    z = jnp.dot(radial, w1_ref[...], preferred_element_type=jnp.float32)
    lane = lax.broadcasted_iota(jnp.int32, (z.shape[0], 128), 1)
    ajidx = lane // _HID
    pat = jnp.where(ajidx == 0, pa[0],
          jnp.where(ajidx == 1, pa[1],
          jnp.where(ajidx == 2, pa[2], pa[3])))

    def silu(x):
        xh = 0.5 * x
        return xh * jnp.tanh(xh) + xh

    outs = (o0, o1, o2, o3)
    for l in range(4):
        n = _N_MAX_L[l]
        hh = silu(z[:, l * 128:(l + 1) * 128] * pat)
        for layer in range(2):
            y = jnp.dot(hh, w23_ref[layer, l], preferred_element_type=jnp.float32)
            hh = silu(y)
        y = jnp.dot(hh, w4_ref[l][:, :4 * n], preferred_element_type=jnp.float32)
        outs[l][...] = y


def _tc_mlp(G, r2, sp2, W1all, W23, W4all, comb_W):
    npad = G.shape[1]
    nb = npad // _BLK
    out_shapes = tuple(jax.ShapeDtypeStruct((npad, 4 * n), jnp.float32)
                       for n in _N_MAX_L)
    return pl.pallas_call(
        _tc_body,
        grid=(nb,),
        in_specs=[
            pl.BlockSpec((2, _BLK, 128), lambda i: (0, i, 0)),
            pl.BlockSpec((_BLK, 1), lambda i: (i, 0)),
            pl.BlockSpec((_BLK, 1), lambda i: (i, 0)),
            pl.BlockSpec((128, 512), lambda i: (0, 0)),
            pl.BlockSpec((2, 4, 128, 128), lambda i: (0, 0, 0, 0)),
            pl.BlockSpec((4, 128, 80), lambda i: (0, 0, 0)),
            pl.BlockSpec(memory_space=pltpu.SMEM),
        ],
        out_specs=tuple(pl.BlockSpec((_BLK, 4 * n), lambda i: (i, 0))
                        for n in _N_MAX_L),
        out_shape=out_shapes,
    )(G, r2, sp2, W1all, W23, W4all, comb_W)


def _pack_weights(mlp_params):
    eye4 = jnp.eye(4, dtype=jnp.float32)
    w1_cols, w23_l, w4_l = [], [[], []], []
    for l in range(4):
        n = _N_MAX_L[l]
        o = _OFFS[l]
        p = [mlp_params[str(l) + '_' + str(aj)] for aj in range(4)]
        w1 = jnp.stack([pi[0] for pi in p])
        w1 = w1.transpose(2, 0, 1).reshape(n, 128)
        w1_cols.append(jnp.pad(w1, ((o, 128 - o - n), (0, 0))))
        for layer in range(2):
            w = jnp.stack([pi[1 + layer].T for pi in p])
            bd = jnp.einsum('aij,ab->aibj', w, eye4).reshape(128, 128)
            w23_l[layer].append(bd)
        w4 = jnp.stack([pi[3].T for pi in p])
        bd = jnp.einsum('aij,ab->aibj', w4, eye4).reshape(128, 4 * n)
        w4_l.append(jnp.pad(bd, ((0, 0), (0, 80 - 4 * n))))
    W1all = jnp.concatenate(w1_cols, axis=1)
    W23 = jnp.stack([jnp.stack(w23_l[0]), jnp.stack(w23_l[1])])
    W4all = jnp.stack(w4_l)
    return W1all, W23, W4all


def _fused_table(spline_values, spline_derivs):
    def slab(x):
        xb = x.astype(jnp.bfloat16)
        u = lax.bitcast_convert_type(xb, jnp.uint16).astype(jnp.uint32)
        return jnp.pad(u, ((0, 0), (0, 128 - _TOTAL_N)))

    sv1 = jnp.concatenate([spline_values[1:], spline_values[:1]], axis=0)
    sd1 = jnp.concatenate([spline_derivs[1:], spline_derivs[:1]], axis=0)
    lo = jnp.concatenate([slab(spline_values), slab(spline_derivs)], axis=1)
    hi = jnp.concatenate([slab(sv1), slab(sd1)], axis=1)
    return lax.bitcast_convert_type((hi << 16) | lo, jnp.int32)


def kernel(r, species_neighbor_idx, spline_values, spline_derivs, comb_W,
           mlp_params):
    n = r.shape[0]
    ftab = _fused_table(spline_values, spline_derivs)
    W1all, W23, W4all = _pack_weights(mlp_params)
    nslice = 2
    quant = int(np.lcm(_NW * 8, _BLK))
    bounds = [(n * j // nslice) // quant * quant for j in range(nslice)] + [n]
    parts = []
    for j in range(nslice):
        sl = slice(bounds[j], bounds[j + 1])
        m = bounds[j + 1] - bounds[j]
        G = _sc_gather(r[sl], ftab)
        outs = _tc_mlp(G, r[sl].reshape(m, 1),
                       species_neighbor_idx[sl].reshape(m, 1),
                       W1all, W23, W4all, comb_W)
        parts.append(tuple(outs[l].reshape(m, 4, _N_MAX_L[l])
                           for l in range(4)))
    return tuple(
        jnp.concatenate([parts[j][l] for j in range(nslice)], axis=0)
        for l in range(4))


# speedup vs baseline: 2.7381x; 1.9607x over previous
import functools

import jax
import jax.numpy as jnp
import numpy as np
from jax import lax
from jax.experimental import pallas as pl
from jax.experimental.pallas import tpu as pltpu
from jax.experimental.pallas import tpu_sc as plsc

_R_CUT = 5.0
_N_POINTS = 600
_N_MAX_L = (20, 18, 16, 14)
_OFFS = (0, 20, 38, 54, 68)
_TOTAL_N = 68
_HID = 32

_NC, _NSUB = 2, 16
_NW = _NC * _NSUB
_CHUNK = 128
_ROW = 256
_BLK = 1280
_ROFF = (0, 24, 48, 64, 80)

_SCALE = np.float32((_N_POINTS - 1) / _R_CUT)
_H = np.float32(_R_CUT / (_N_POINTS - 1))


def _sc_gather(r, ftab):
    n = r.shape[0]
    per_w = n // _NW
    n_chunks = (per_w + _CHUNK - 1) // _CHUNK
    last_off = per_w - _CHUNK
    mesh = plsc.VectorSubcoreMesh(core_axis_name="c", subcore_axis_name="s")

    @functools.partial(
        pl.kernel,
        out_type=jax.ShapeDtypeStruct((2, n, 128), jnp.int32),
        mesh=mesh,
        scratch_types=[
            pltpu.VMEM((_CHUNK,), jnp.float32),
            pltpu.VMEM((_CHUNK,), jnp.int32),
            pltpu.VMEM((_CHUNK,), jnp.int32),
            pltpu.VMEM((_CHUNK, _ROW), jnp.int32),
            pltpu.VMEM((_CHUNK, _ROW), jnp.int32),
            pltpu.SemaphoreType.DMA,
            pltpu.SemaphoreType.DMA,
        ],
    )
    def k(r_hbm, tab_hbm, out_hbm, r_v, idx0, idx1, rows0, rows1, sem0, sem1):
        wid = lax.axis_index("s") * _NC + lax.axis_index("c")
        base = wid * per_w

        def chunk_off(c):
            return base + jnp.minimum(c * _CHUNK, last_off)

        def load_idx(c, idx_v):
            off = chunk_off(c)
            pltpu.sync_copy(r_hbm.at[pl.ds(off, _CHUNK)], r_v)
            for kk in range(_CHUNK // 16):
                rv = r_v[pl.ds(kk * 16, 16)]
                i0 = (rv * _SCALE).astype(jnp.int32)
                i0 = jnp.minimum(jnp.maximum(i0, 0), _N_POINTS - 2)
                idx_v[pl.ds(kk * 16, 16)] = i0

        bufs = ((idx0, rows0, sem0), (idx1, rows1, sem1))

        for b in range(2):
            idx_v, rows_v, sem = bufs[b]
            load_idx(b, idx_v)
            pltpu.async_copy(tab_hbm.at[idx_v], rows_v, sem)

        def body(i, carry):
            for b in range(2):
                c = 2 * i + b
                idx_v, rows_v, sem = bufs[b]
                pltpu.make_async_copy(tab_hbm.at[idx_v], rows_v, sem).wait()
                off = chunk_off(c)
                pltpu.sync_copy(rows_v.at[:, pl.ds(0, 128)],
                                out_hbm.at[0, pl.ds(off, _CHUNK)])
                pltpu.sync_copy(rows_v.at[:, pl.ds(128, 128)],
                                out_hbm.at[1, pl.ds(off, _CHUNK)])

                @pl.when(c + 2 < n_chunks)
                def _():
                    load_idx(c + 2, idx_v)
                    pltpu.async_copy(tab_hbm.at[idx_v], rows_v, sem)

            return carry

        lax.fori_loop(0, (n_chunks + 1) // 2, body, 0)

    return k(r, ftab)


def _tc_body(g_ref, r_ref, s_ref, w1t_ref, w23t_ref, w4t_ref, comb_ref,
             o0, o1, o2, o3):
    gw0 = jnp.transpose(g_ref[0])
    gw1 = jnp.transpose(g_ref[1])
    mask = jnp.int32(-65536)

    def unpack(gw):
        lo = lax.bitcast_convert_type(jnp.left_shift(gw, 16), jnp.float32)
        hi = lax.bitcast_convert_type(jnp.bitwise_and(gw, mask), jnp.float32)
        return lo, hi

    v0, v1 = unpack(gw0)
    d0, d1 = unpack(gw1)
    t = r_ref[0] * _SCALE
    i0 = t.astype(jnp.int32)
    i0 = jnp.minimum(jnp.maximum(i0, 0), _N_POINTS - 2)
    f = t - i0.astype(jnp.float32)
    f2 = f * f
    f3 = f2 * f
    h00 = 2.0 * f3 - 3.0 * f2 + 1.0
    h10 = _H * (f3 - 2.0 * f2 + f)
    h01 = -2.0 * f3 + 3.0 * f2
    h11 = _H * (f3 - f2)
    radial = h00 * v0 + h10 * d0 + h01 * v1 + h11 * d1

    s = s_ref[0]
    pa = []
    for aj in range(4):
        v = jnp.where(s == 0, comb_ref[aj, 0],
            jnp.where(s == 1, comb_ref[aj, 1],
            jnp.where(s == 2, comb_ref[aj, 2], comb_ref[aj, 3])))
        pa.append(v)

    def silu(x):
        xh = 0.5 * x
        return xh * jnp.tanh(xh) + xh

    outs = (o0, o1, o2, o3)
    nq = radial.shape[1] // 256
    for aj in range(4):
        hh = silu(jnp.dot(w1t_ref[aj], radial * pa[aj],
                          preferred_element_type=jnp.float32))
        for layer in range(2):
            hh = silu(jnp.dot(w23t_ref[layer, aj], hh,
                              preferred_element_type=jnp.float32))
        y = jnp.dot(w4t_ref[aj], hh, preferred_element_type=jnp.float32)
        for l in range(4):
            n = _N_MAX_L[l]
            yl = y[_ROFF[l]:_ROFF[l] + n]
            y4 = yl.reshape(n, nq, 2, 128)
            outs[l][:, :, aj, :] = y4[:, :, 0, :]
            outs[l][:, :, aj + 4, :] = y4[:, :, 1, :]


def _tc_mlp(G, r3, sp3, W1T, W23T, W4T, comb_W):
    npad = G.shape[1]
    nb = npad // _BLK
    nq = _BLK // 256
    out_shapes = tuple(
        jax.ShapeDtypeStruct((n, npad // 256, 8, 128), jnp.float32)
        for n in _N_MAX_L)
    return pl.pallas_call(
        _tc_body,
        grid=(nb,),
        in_specs=[
            pl.BlockSpec((2, _BLK, 128), lambda i: (0, i, 0)),
            pl.BlockSpec((1, 1, _BLK), lambda i: (i, 0, 0)),
            pl.BlockSpec((1, 1, _BLK), lambda i: (i, 0, 0)),
            pl.BlockSpec((4, 128, 128), lambda i: (0, 0, 0)),
            pl.BlockSpec((2, 4, 128, 128), lambda i: (0, 0, 0, 0)),
            pl.BlockSpec((4, 80, 128), lambda i: (0, 0, 0)),
            pl.BlockSpec(memory_space=pltpu.SMEM),
        ],
        out_specs=tuple(pl.BlockSpec((n, nq, 8, 128), lambda i: (0, i, 0, 0))
                        for n in _N_MAX_L),
        out_shape=out_shapes,
    )(G, r3, sp3, W1T, W23T, W4T, comb_W)


def _pack_weights(mlp_params):
    eye4 = jnp.eye(4, dtype=jnp.float32)
    w1_aj, w23_aj, w4_aj = [], [[], []], []
    for aj in range(4):
        p = [mlp_params[str(l) + '_' + str(aj)] for l in range(4)]
        blocks = [jnp.pad(p[l][0],
                          ((0, 0), (_OFFS[l], 128 - _OFFS[l] - _N_MAX_L[l])))
                  for l in range(4)]
        w1_aj.append(jnp.concatenate(blocks, axis=0))
        for layer in range(2):
            S = jnp.stack([p[l][1 + layer] for l in range(4)])
            bd = jnp.einsum('lij,lm->limj', S, eye4).reshape(128, 128)
            w23_aj[layer].append(bd)
        blocks = []
        for l in range(4):
            n = _N_MAX_L[l]
            rows = _ROFF[l + 1] - _ROFF[l]
            b = jnp.pad(p[l][3], ((0, rows - n), (l * 32, 96 - l * 32)))
            blocks.append(b)
        w4_aj.append(jnp.concatenate(blocks, axis=0))
    W1T = jnp.stack(w1_aj)
    W23T = jnp.stack([jnp.stack(w23_aj[0]), jnp.stack(w23_aj[1])])
    W4T = jnp.stack(w4_aj)
    return W1T, W23T, W4T


def _fused_table(spline_values, spline_derivs):
    def slab(x):
        xb = x.astype(jnp.bfloat16)
        u = lax.bitcast_convert_type(xb, jnp.uint16).astype(jnp.uint32)
        return jnp.pad(u, ((0, 0), (0, 128 - _TOTAL_N)))

    sv1 = jnp.concatenate([spline_values[1:], spline_values[:1]], axis=0)
    sd1 = jnp.concatenate([spline_derivs[1:], spline_derivs[:1]], axis=0)
    lo = jnp.concatenate([slab(spline_values), slab(spline_derivs)], axis=1)
    hi = jnp.concatenate([slab(sv1), slab(sd1)], axis=1)
    return lax.bitcast_convert_type((hi << 16) | lo, jnp.int32)


def kernel(r, species_neighbor_idx, spline_values, spline_derivs, comb_W,
           mlp_params):
    n = r.shape[0]
    ftab = _fused_table(spline_values, spline_derivs)
    W1T, W23T, W4T = _pack_weights(mlp_params)
    G = _sc_gather(r, ftab)
    nb = n // _BLK
    outs = _tc_mlp(G, r.reshape(nb, 1, _BLK),
                   species_neighbor_idx.reshape(nb, 1, _BLK),
                   W1T, W23T, W4T, comb_W)
    res = []
    for l in range(4):
        nl = _N_MAX_L[l]
        t = outs[l].reshape(nl, n // 256, 2, 4, 128)
        res.append(t.transpose(1, 2, 4, 3, 0).reshape(n, 4, nl))
    return tuple(res)


# speedup vs baseline: 3.2469x; 1.1858x over previous
import functools

import jax
import jax.numpy as jnp
import numpy as np
from jax import lax
from jax.experimental import pallas as pl
from jax.experimental.pallas import tpu as pltpu
from jax.experimental.pallas import tpu_sc as plsc

_R_CUT = 5.0
_N_POINTS = 600
_N_MAX_L = (20, 18, 16, 14)
_OFFS = (0, 20, 38, 54, 68)
_TOTAL_N = 68
_HID = 32

_NC, _NSUB = 2, 16
_NW = _NC * _NSUB
_CHUNK = 128
_ROW = 256
_BLK = 1280
_ROFF = (0, 24, 48, 64, 80)

_SCALE = np.float32((_N_POINTS - 1) / _R_CUT)
_H = np.float32(_R_CUT / (_N_POINTS - 1))


def _sc_gather(r, ftab):
    n = r.shape[0]
    per_w = n // _NW
    n_chunks = (per_w + _CHUNK - 1) // _CHUNK
    last_off = per_w - _CHUNK
    mesh = plsc.VectorSubcoreMesh(core_axis_name="c", subcore_axis_name="s")

    @functools.partial(
        pl.kernel,
        out_type=jax.ShapeDtypeStruct((2, n, 128), jnp.int32),
        mesh=mesh,
        scratch_types=[
            pltpu.VMEM((_CHUNK,), jnp.float32),
            pltpu.VMEM((_CHUNK,), jnp.int32),
            pltpu.VMEM((_CHUNK,), jnp.int32),
            pltpu.VMEM((_CHUNK, _ROW), jnp.int32),
            pltpu.VMEM((_CHUNK, _ROW), jnp.int32),
            pltpu.SemaphoreType.DMA,
            pltpu.SemaphoreType.DMA,
        ],
    )
    def k(r_hbm, tab_hbm, out_hbm, r_v, idx0, idx1, rows0, rows1, sem0, sem1):
        wid = lax.axis_index("s") * _NC + lax.axis_index("c")
        base = wid * per_w

        def chunk_off(c):
            return base + jnp.minimum(c * _CHUNK, last_off)

        def load_idx(c, idx_v):
            off = chunk_off(c)
            pltpu.sync_copy(r_hbm.at[pl.ds(off, _CHUNK)], r_v)
            for kk in range(_CHUNK // 16):
                rv = r_v[pl.ds(kk * 16, 16)]
                i0 = (rv * _SCALE).astype(jnp.int32)
                i0 = jnp.minimum(jnp.maximum(i0, 0), _N_POINTS - 2)
                idx_v[pl.ds(kk * 16, 16)] = i0

        bufs = ((idx0, rows0, sem0), (idx1, rows1, sem1))

        for b in range(2):
            idx_v, rows_v, sem = bufs[b]
            load_idx(b, idx_v)
            pltpu.async_copy(tab_hbm.at[idx_v], rows_v, sem)

        def body(i, carry):
            for b in range(2):
                c = 2 * i + b
                idx_v, rows_v, sem = bufs[b]
                pltpu.make_async_copy(tab_hbm.at[idx_v], rows_v, sem).wait()
                off = chunk_off(c)
                pltpu.sync_copy(rows_v.at[:, pl.ds(0, 128)],
                                out_hbm.at[0, pl.ds(off, _CHUNK)])
                pltpu.sync_copy(rows_v.at[:, pl.ds(128, 128)],
                                out_hbm.at[1, pl.ds(off, _CHUNK)])

                @pl.when(c + 2 < n_chunks)
                def _():
                    load_idx(c + 2, idx_v)
                    pltpu.async_copy(tab_hbm.at[idx_v], rows_v, sem)

            return carry

        lax.fori_loop(0, (n_chunks + 1) // 2, body, 0)

    return k(r, ftab)


def _tc_body(g_ref, r_ref, s_ref, w1t_ref, w23t_ref, w4t_ref, comb_ref,
             o0, o1, o2, o3):
    gw0 = jnp.transpose(g_ref[0])
    gw1 = jnp.transpose(g_ref[1])
    mask = jnp.int32(-65536)

    def unpack(gw):
        lo = lax.bitcast_convert_type(jnp.left_shift(gw, 16), jnp.float32)
        hi = lax.bitcast_convert_type(jnp.bitwise_and(gw, mask), jnp.float32)
        return lo, hi

    v0, v1 = unpack(gw0)
    d0, d1 = unpack(gw1)
    t = r_ref[0] * _SCALE
    i0 = t.astype(jnp.int32)
    i0 = jnp.minimum(jnp.maximum(i0, 0), _N_POINTS - 2)
    f = t - i0.astype(jnp.float32)
    f2 = f * f
    f3 = f2 * f
    h00 = 2.0 * f3 - 3.0 * f2 + 1.0
    h10 = _H * (f3 - 2.0 * f2 + f)
    h01 = -2.0 * f3 + 3.0 * f2
    h11 = _H * (f3 - f2)
    radial = h00 * v0 + h10 * d0 + h01 * v1 + h11 * d1

    s = s_ref[0]
    pa = []
    for aj in range(4):
        v = jnp.where(s == 0, comb_ref[aj, 0],
            jnp.where(s == 1, comb_ref[aj, 1],
            jnp.where(s == 2, comb_ref[aj, 2], comb_ref[aj, 3])))
        pa.append(v)

    def silu(x):
        xh = 0.5 * x
        return xh * jnp.tanh(xh) + xh

    outs = (o0, o1, o2, o3)
    nq = radial.shape[1] // 256
    for aj in range(4):
        hh = silu(jnp.dot(w1t_ref[aj], radial * pa[aj],
                          preferred_element_type=jnp.float32))
        for layer in range(2):
            hh = silu(jnp.dot(w23t_ref[layer, aj], hh,
                              preferred_element_type=jnp.float32))
        y = jnp.dot(w4t_ref[aj], hh, preferred_element_type=jnp.float32)
        for l in range(4):
            n = _N_MAX_L[l]
            yl = y[_ROFF[l]:_ROFF[l] + n]
            if l == 2:
                y5 = yl.reshape(2, 8, 2 * nq, 128)
                for nbl in range(2 * nq):
                    outs[l][aj, :, nbl] = y5[:, :, nbl, :]
            else:
                y4 = yl.reshape(n, nq, 2, 128)
                outs[l][:, :, aj, :] = y4[:, :, 0, :]
                outs[l][:, :, aj + 4, :] = y4[:, :, 1, :]


def _tc_mlp(G, r3, sp3, W1T, W23T, W4T, comb_W):
    npad = G.shape[1]
    nb = npad // _BLK
    nq = _BLK // 256
    out_shapes = tuple(
        jax.ShapeDtypeStruct((4, 2, npad // 128, 8, 128), jnp.float32)
        if l == 2 else
        jax.ShapeDtypeStruct((_N_MAX_L[l], npad // 256, 8, 128), jnp.float32)
        for l in range(4))
    return pl.pallas_call(
        _tc_body,
        grid=(nb,),
        in_specs=[
            pl.BlockSpec((2, _BLK, 128), lambda i: (0, i, 0)),
            pl.BlockSpec((1, 1, _BLK), lambda i: (i, 0, 0)),
            pl.BlockSpec((1, 1, _BLK), lambda i: (i, 0, 0)),
            pl.BlockSpec((4, 128, 128), lambda i: (0, 0, 0)),
            pl.BlockSpec((2, 4, 128, 128), lambda i: (0, 0, 0, 0)),
            pl.BlockSpec((4, 80, 128), lambda i: (0, 0, 0)),
            pl.BlockSpec(memory_space=pltpu.SMEM),
        ],
        out_specs=tuple(
            pl.BlockSpec((4, 2, 2 * nq, 8, 128), lambda i: (0, 0, i, 0, 0))
            if l == 2 else
            pl.BlockSpec((_N_MAX_L[l], nq, 8, 128), lambda i: (0, i, 0, 0))
            for l in range(4)),
        out_shape=out_shapes,
    )(G, r3, sp3, W1T, W23T, W4T, comb_W)


def _pack_weights(mlp_params):
    eye4 = jnp.eye(4, dtype=jnp.float32)
    w1_aj, w23_aj, w4_aj = [], [[], []], []
    for aj in range(4):
        p = [mlp_params[str(l) + '_' + str(aj)] for l in range(4)]
        blocks = [jnp.pad(p[l][0],
                          ((0, 0), (_OFFS[l], 128 - _OFFS[l] - _N_MAX_L[l])))
                  for l in range(4)]
        w1_aj.append(jnp.concatenate(blocks, axis=0))
        for layer in range(2):
            S = jnp.stack([p[l][1 + layer] for l in range(4)])
            bd = jnp.einsum('lij,lm->limj', S, eye4).reshape(128, 128)
            w23_aj[layer].append(bd)
        blocks = []
        for l in range(4):
            n = _N_MAX_L[l]
            rows = _ROFF[l + 1] - _ROFF[l]
            b = jnp.pad(p[l][3], ((0, rows - n), (l * 32, 96 - l * 32)))
            blocks.append(b)
        w4_aj.append(jnp.concatenate(blocks, axis=0))
    W1T = jnp.stack(w1_aj)
    W23T = jnp.stack([jnp.stack(w23_aj[0]), jnp.stack(w23_aj[1])])
    W4T = jnp.stack(w4_aj)
    return W1T, W23T, W4T


def _fused_table(spline_values, spline_derivs):
    def slab(x):
        xb = x.astype(jnp.bfloat16)
        u = lax.bitcast_convert_type(xb, jnp.uint16).astype(jnp.uint32)
        return jnp.pad(u, ((0, 0), (0, 128 - _TOTAL_N)))

    sv1 = jnp.concatenate([spline_values[1:], spline_values[:1]], axis=0)
    sd1 = jnp.concatenate([spline_derivs[1:], spline_derivs[:1]], axis=0)
    lo = jnp.concatenate([slab(spline_values), slab(spline_derivs)], axis=1)
    hi = jnp.concatenate([slab(sv1), slab(sd1)], axis=1)
    return lax.bitcast_convert_type((hi << 16) | lo, jnp.int32)


def kernel(r, species_neighbor_idx, spline_values, spline_derivs, comb_W,
           mlp_params):
    n = r.shape[0]
    ftab = _fused_table(spline_values, spline_derivs)
    W1T, W23T, W4T = _pack_weights(mlp_params)
    G = _sc_gather(r, ftab)
    nb = n // _BLK
    outs = _tc_mlp(G, r.reshape(nb, 1, _BLK),
                   species_neighbor_idx.reshape(nb, 1, _BLK),
                   W1T, W23T, W4T, comb_W)
    res = []
    for l in range(4):
        nl = _N_MAX_L[l]
        if l == 2:
            t = outs[l].transpose(2, 4, 0, 1, 3)
            res.append(t.reshape(n, 4, nl))
        else:
            t = outs[l].reshape(nl, n // 256, 2, 4, 128)
            res.append(t.transpose(1, 2, 4, 3, 0).reshape(n, 4, nl))
    return tuple(res)


# speedup vs baseline: 3.2986x; 1.0159x over previous
import functools

import jax
import jax.numpy as jnp
import numpy as np
from jax import lax
from jax.experimental import pallas as pl
from jax.experimental.pallas import tpu as pltpu
from jax.experimental.pallas import tpu_sc as plsc

_R_CUT = 5.0
_N_POINTS = 600
_N_MAX_L = (20, 18, 16, 14)
_OFFS = (0, 20, 38, 54, 68)
_TOTAL_N = 68
_HID = 32

_NC, _NSUB = 2, 16
_NW = _NC * _NSUB
_CHUNK = 128
_ROW = 256
_BLK = 1280
_ROFF = (0, 24, 48, 64, 80)

_SCALE = np.float32((_N_POINTS - 1) / _R_CUT)
_H = np.float32(_R_CUT / (_N_POINTS - 1))


def _sc_gather(r, ftab):
    n = r.shape[0]
    per_w = n // _NW
    n_chunks = (per_w + _CHUNK - 1) // _CHUNK
    last_off = per_w - _CHUNK
    mesh = plsc.VectorSubcoreMesh(core_axis_name="c", subcore_axis_name="s")

    @functools.partial(
        pl.kernel,
        out_type=jax.ShapeDtypeStruct((2, n, 128), jnp.int32),
        mesh=mesh,
        scratch_types=[
            pltpu.VMEM((_CHUNK,), jnp.float32),
            pltpu.VMEM((_CHUNK,), jnp.int32),
            pltpu.VMEM((_CHUNK,), jnp.int32),
            pltpu.VMEM((_CHUNK, _ROW), jnp.int32),
            pltpu.VMEM((_CHUNK, _ROW), jnp.int32),
            pltpu.SemaphoreType.DMA,
            pltpu.SemaphoreType.DMA,
        ],
    )
    def k(r_hbm, tab_hbm, out_hbm, r_v, idx0, idx1, rows0, rows1, sem0, sem1):
        wid = lax.axis_index("s") * _NC + lax.axis_index("c")
        base = wid * per_w

        def chunk_off(c):
            return base + jnp.minimum(c * _CHUNK, last_off)

        def load_idx(c, idx_v):
            off = chunk_off(c)
            pltpu.sync_copy(r_hbm.at[pl.ds(off, _CHUNK)], r_v)
            for kk in range(_CHUNK // 16):
                rv = r_v[pl.ds(kk * 16, 16)]
                i0 = (rv * _SCALE).astype(jnp.int32)
                i0 = jnp.minimum(jnp.maximum(i0, 0), _N_POINTS - 2)
                idx_v[pl.ds(kk * 16, 16)] = i0

        bufs = ((idx0, rows0, sem0), (idx1, rows1, sem1))

        for b in range(2):
            idx_v, rows_v, sem = bufs[b]
            load_idx(b, idx_v)
            pltpu.async_copy(tab_hbm.at[idx_v], rows_v, sem)

        def body(i, carry):
            for b in range(2):
                c = 2 * i + b
                idx_v, rows_v, sem = bufs[b]
                pltpu.make_async_copy(tab_hbm.at[idx_v], rows_v, sem).wait()
                off = chunk_off(c)
                pltpu.sync_copy(rows_v.at[:, pl.ds(0, 128)],
                                out_hbm.at[0, pl.ds(off, _CHUNK)])
                pltpu.sync_copy(rows_v.at[:, pl.ds(128, 128)],
                                out_hbm.at[1, pl.ds(off, _CHUNK)])

                @pl.when(c + 2 < n_chunks)
                def _():
                    load_idx(c + 2, idx_v)
                    pltpu.async_copy(tab_hbm.at[idx_v], rows_v, sem)

            return carry

        lax.fori_loop(0, (n_chunks + 1) // 2, body, 0)

    return k(r, ftab)


def _tc_body(g_ref, r_ref, s_ref, w1t_ref, w23t_ref, w4t_ref, comb_ref,
             o0, o1, o2, o3):
    gw0 = jnp.transpose(g_ref[0])
    gw1 = jnp.transpose(g_ref[1])
    mask = jnp.int32(-65536)

    def unpack(gw):
        lo = lax.bitcast_convert_type(jnp.left_shift(gw, 16), jnp.float32)
        hi = lax.bitcast_convert_type(jnp.bitwise_and(gw, mask), jnp.float32)
        return lo, hi

    v0, v1 = unpack(gw0)
    d0, d1 = unpack(gw1)
    t = r_ref[0] * _SCALE
    i0 = t.astype(jnp.int32)
    i0 = jnp.minimum(jnp.maximum(i0, 0), _N_POINTS - 2)
    f = t - i0.astype(jnp.float32)
    f2 = f * f
    f3 = f2 * f
    h00 = 2.0 * f3 - 3.0 * f2 + 1.0
    h10 = _H * (f3 - 2.0 * f2 + f)
    h01 = -2.0 * f3 + 3.0 * f2
    h11 = _H * (f3 - f2)
    radial = h00 * v0 + h10 * d0 + h01 * v1 + h11 * d1

    s = s_ref[0]
    pa = []
    for aj in range(4):
        v = jnp.where(s == 0, comb_ref[aj, 0],
            jnp.where(s == 1, comb_ref[aj, 1],
            jnp.where(s == 2, comb_ref[aj, 2], comb_ref[aj, 3])))
        pa.append(v)

    def silu(x):
        xh = 0.5 * x
        return xh * jnp.tanh(xh) + xh

    outs = (o0, o1, o2, o3)
    nq = radial.shape[1] // 256
    for aj in range(4):
        hh = silu(jnp.dot(w1t_ref[aj], radial * pa[aj],
                          preferred_element_type=jnp.float32))
        for layer in range(2):
            hh = silu(jnp.dot(w23t_ref[layer, aj], hh,
                              preferred_element_type=jnp.float32))
        y = jnp.dot(w4t_ref[aj], hh, preferred_element_type=jnp.float32)
        for l in range(4):
            n = _N_MAX_L[l]
            yl = y[_ROFF[l]:_ROFF[l] + n]
            if l == 2:
                for tj in range(2):
                    for nbl in range(2 * nq):
                        outs[l][aj, tj, nbl] = (
                            yl[tj * 8:tj * 8 + 8, nbl * 128:nbl * 128 + 128])
            else:
                for qq in range(nq):
                    outs[l][:, qq, aj, :] = (
                        yl[:, 2 * qq * 128:2 * qq * 128 + 128])
                    outs[l][:, qq, aj + 4, :] = (
                        yl[:, (2 * qq + 1) * 128:(2 * qq + 1) * 128 + 128])


def _tc_mlp(G, r3, sp3, W1T, W23T, W4T, comb_W):
    npad = G.shape[1]
    nb = npad // _BLK
    nq = _BLK // 256
    out_shapes = tuple(
        jax.ShapeDtypeStruct((4, 2, npad // 128, 8, 128), jnp.float32)
        if l == 2 else
        jax.ShapeDtypeStruct((_N_MAX_L[l], npad // 256, 8, 128), jnp.float32)
        for l in range(4))
    return pl.pallas_call(
        _tc_body,
        grid=(nb,),
        in_specs=[
            pl.BlockSpec((2, _BLK, 128), lambda i: (0, i, 0)),
            pl.BlockSpec((1, 1, _BLK), lambda i: (i, 0, 0)),
            pl.BlockSpec((1, 1, _BLK), lambda i: (i, 0, 0)),
            pl.BlockSpec((4, 128, 128), lambda i: (0, 0, 0)),
            pl.BlockSpec((2, 4, 128, 128), lambda i: (0, 0, 0, 0)),
            pl.BlockSpec((4, 80, 128), lambda i: (0, 0, 0)),
            pl.BlockSpec(memory_space=pltpu.SMEM),
        ],
        out_specs=tuple(
            pl.BlockSpec((4, 2, 2 * nq, 8, 128), lambda i: (0, 0, i, 0, 0))
            if l == 2 else
            pl.BlockSpec((_N_MAX_L[l], nq, 8, 128), lambda i: (0, i, 0, 0))
            for l in range(4)),
        out_shape=out_shapes,
    )(G, r3, sp3, W1T, W23T, W4T, comb_W)


def _pack_weights(mlp_params):
    eye4 = jnp.eye(4, dtype=jnp.float32)
    w1_aj, w23_aj, w4_aj = [], [[], []], []
    for aj in range(4):
        p = [mlp_params[str(l) + '_' + str(aj)] for l in range(4)]
        blocks = [jnp.pad(p[l][0],
                          ((0, 0), (_OFFS[l], 128 - _OFFS[l] - _N_MAX_L[l])))
                  for l in range(4)]
        w1_aj.append(jnp.concatenate(blocks, axis=0))
        for layer in range(2):
            S = jnp.stack([p[l][1 + layer] for l in range(4)])
            bd = jnp.einsum('lij,lm->limj', S, eye4).reshape(128, 128)
            w23_aj[layer].append(bd)
        blocks = []
        for l in range(4):
            n = _N_MAX_L[l]
            rows = _ROFF[l + 1] - _ROFF[l]
            b = jnp.pad(p[l][3], ((0, rows - n), (l * 32, 96 - l * 32)))
            blocks.append(b)
        w4_aj.append(jnp.concatenate(blocks, axis=0))
    W1T = jnp.stack(w1_aj)
    W23T = jnp.stack([jnp.stack(w23_aj[0]), jnp.stack(w23_aj[1])])
    W4T = jnp.stack(w4_aj)
    return W1T, W23T, W4T


def _fused_table(spline_values, spline_derivs):
    def slab(x):
        xb = x.astype(jnp.bfloat16)
        u = lax.bitcast_convert_type(xb, jnp.uint16).astype(jnp.uint32)
        return jnp.pad(u, ((0, 0), (0, 128 - _TOTAL_N)))

    sv1 = jnp.concatenate([spline_values[1:], spline_values[:1]], axis=0)
    sd1 = jnp.concatenate([spline_derivs[1:], spline_derivs[:1]], axis=0)
    lo = jnp.concatenate([slab(spline_values), slab(spline_derivs)], axis=1)
    hi = jnp.concatenate([slab(sv1), slab(sd1)], axis=1)
    return lax.bitcast_convert_type((hi << 16) | lo, jnp.int32)


def kernel(r, species_neighbor_idx, spline_values, spline_derivs, comb_W,
           mlp_params):
    n = r.shape[0]
    ftab = _fused_table(spline_values, spline_derivs)
    W1T, W23T, W4T = _pack_weights(mlp_params)
    G = _sc_gather(r, ftab)
    nb = n // _BLK
    outs = _tc_mlp(G, r.reshape(nb, 1, _BLK),
                   species_neighbor_idx.reshape(nb, 1, _BLK),
                   W1T, W23T, W4T, comb_W)
    res = []
    for l in range(4):
        nl = _N_MAX_L[l]
        if l == 2:
            t = outs[l].transpose(2, 4, 0, 1, 3)
            res.append(t.reshape(n, 4, nl))
        else:
            t = outs[l].reshape(nl, n // 256, 2, 4, 128)
            res.append(t.transpose(1, 2, 4, 3, 0).reshape(n, 4, nl))
    return tuple(res)
